# baseline (speedup 1.0000x reference)
import jax
import jax.numpy as jnp
from jax.experimental import pallas as pl
from jax.experimental.pallas import tpu as pltpu

_CHUNK = 4
_NBUF = 6


def _body(w_hbm, o_hbm, inb, outb, insem, outsem):
    steps = w_hbm.shape[0] // _CHUNK

    def start_in(i):
        b = i % _NBUF
        pltpu.make_async_copy(
            w_hbm.at[pl.ds(i * _CHUNK, _CHUNK)], inb.at[b], insem.at[b]
        ).start()

    def wait_in(i):
        b = i % _NBUF
        pltpu.make_async_copy(
            w_hbm.at[pl.ds(i * _CHUNK, _CHUNK)], inb.at[b], insem.at[b]
        ).wait()

    def start_out(i):
        b = i % _NBUF
        pltpu.make_async_copy(
            outb.at[b], o_hbm.at[pl.ds(i * _CHUNK, _CHUNK)], outsem.at[b]
        ).start()

    def wait_out(i):
        b = i % _NBUF
        pltpu.make_async_copy(
            outb.at[b], o_hbm.at[pl.ds(i * _CHUNK, _CHUNK)], outsem.at[b]
        ).wait()

    for i in range(min(_NBUF, steps)):
        start_in(i)
    for i in range(steps):
        b = i % _NBUF
        wait_in(i)
        if i >= _NBUF:
            wait_out(i - _NBUF)
        x = jnp.clip(inb[b], 0.0, 1.0)
        outb[b] = jnp.round(x * 255.0) / 255.0
        start_out(i)
        if i + _NBUF < steps:
            start_in(i + _NBUF)
    for i in range(max(steps - _NBUF, 0), steps):
        wait_out(i)


def kernel(watermark_samples, response):
    n, c, h, w = watermark_samples.shape
    out = pl.pallas_call(
        _body,
        in_specs=[pl.BlockSpec(memory_space=pltpu.HBM)],
        out_specs=pl.BlockSpec(memory_space=pltpu.HBM),
        out_shape=jax.ShapeDtypeStruct((n, c, h, w), jnp.float32),
        scratch_shapes=[
            pltpu.VMEM((_NBUF, _CHUNK, c, h, w), jnp.float32),
            pltpu.VMEM((_NBUF, _CHUNK, c, h, w), jnp.float32),
            pltpu.SemaphoreType.DMA((_NBUF,)),
            pltpu.SemaphoreType.DMA((_NBUF,)),
        ],
    )(watermark_samples)
    return (out, response)


# speedup vs baseline: 4.1840x; 4.1840x over previous
import jax
import jax.numpy as jnp
from jax.experimental import pallas as pl

_BLOCK_H = 16


def _discretize_body(w_ref, o_ref):
    x = jnp.clip(w_ref[...], 0.0, 1.0)
    o_ref[...] = jnp.round(x * 255.0) / 255.0


def kernel(watermark_samples, response):
    n, c, h, w = watermark_samples.shape
    t = jnp.transpose(watermark_samples, (1, 2, 3, 0))
    out = pl.pallas_call(
        _discretize_body,
        grid=(c, h // _BLOCK_H),
        in_specs=[pl.BlockSpec((1, _BLOCK_H, w, n), lambda i, j: (i, j, 0, 0))],
        out_specs=pl.BlockSpec((1, _BLOCK_H, w, n), lambda i, j: (i, j, 0, 0)),
        out_shape=jax.ShapeDtypeStruct((c, h, w, n), jnp.float32),
    )(t)
    return (jnp.transpose(out, (3, 0, 1, 2)), response)
